# Initial kernel scaffold; baseline (speedup 1.0000x reference)
import jax
import jax.numpy as jnp
from jax.experimental import pallas as pl


def kernel(z, embedding):
    raise NotImplementedError("write your pallas kernel here")


# speedup vs baseline: 1.3173x; 1.3173x over previous
import functools

import jax
import jax.numpy as jnp
from jax import lax
from jax.experimental import pallas as pl
from jax.experimental.pallas import tpu as pltpu
from jax.experimental.pallas import tpu_sc as plsc

CODEBOOK_SIZE = 8192
LATENT_DIM = 256
COMMITMENT_COST = 0.25

_TB = 512
_KB = 2048


def _argmin_body(z_ref, e_ref, ids_ref, loss_ref):
    t = pl.program_id(0)
    nt = pl.num_programs(0)
    z = z_ref[...]
    zn = jnp.sum(z * z, axis=1, keepdims=True)

    best_val = jnp.full((_TB, 1), jnp.inf, dtype=jnp.float32)
    best_idx = jnp.zeros((_TB, 1), dtype=jnp.int32)
    for kb in range(CODEBOOK_SIZE // _KB):
        e = e_ref[pl.ds(kb * _KB, _KB), :]
        en = jnp.sum(e * e, axis=1)
        zw = lax.dot_general(z, e, (((1,), (1,)), ((), ())),
                             preferred_element_type=jnp.float32)
        d = (zn + en[None, :]) - 2.0 * zw
        m = jnp.min(d, axis=1, keepdims=True)
        iota = lax.broadcasted_iota(jnp.int32, (_TB, _KB), 1) + kb * _KB
        idx = jnp.min(jnp.where(d == m, iota, CODEBOOK_SIZE),
                      axis=1, keepdims=True)
        better = m < best_val
        best_idx = jnp.where(better, idx, best_idx)
        best_val = jnp.where(better, m, best_val)

    ids_ref[...] = best_idx.reshape(1, 1, _TB)

    part = jnp.sum(best_val).reshape(1, 1)

    @pl.when(t == 0)
    def _init():
        loss_ref[...] = jnp.zeros((1, 1), jnp.float32)

    loss_ref[...] += part

    @pl.when(t == nt - 1)
    def _fin():
        n_elems = jnp.float32(nt * _TB * LATENT_DIM)
        loss_ref[...] = loss_ref[...] * ((1.0 + COMMITMENT_COST) / n_elems)


def _tc_argmin(z_flat, embedding):
    n_tok, d_dim = z_flat.shape
    nt = n_tok // _TB
    return pl.pallas_call(
        _argmin_body,
        grid=(nt,),
        in_specs=[
            pl.BlockSpec((_TB, d_dim), lambda t: (t, 0)),
            pl.BlockSpec((CODEBOOK_SIZE, d_dim), lambda t: (0, 0)),
        ],
        out_specs=[
            pl.BlockSpec((1, 1, _TB), lambda t: (t, 0, 0)),
            pl.BlockSpec((1, 1), lambda t: (0, 0)),
        ],
        out_shape=[
            jax.ShapeDtypeStruct((nt, 1, _TB), jnp.int32),
            jax.ShapeDtypeStruct((1, 1), jnp.float32),
        ],
    )(z_flat, embedding)


def _sc_gather(embedding, ids_flat):
    n_tok = ids_flat.shape[0]
    d_dim = embedding.shape[1]
    info = plsc.get_sparse_core_info()
    nc, ns = info.num_cores, info.num_subcores
    nw = nc * ns
    b_per_w = n_tok // nw
    n_chunks = 2
    chunk = b_per_w // n_chunks
    mesh = plsc.VectorSubcoreMesh(core_axis_name="c", subcore_axis_name="s")

    @functools.partial(
        pl.kernel,
        mesh=mesh,
        out_type=jax.ShapeDtypeStruct((n_tok, d_dim), jnp.float32),
        scratch_types=[
            pltpu.VMEM((chunk,), jnp.int32),
            pltpu.VMEM((chunk, d_dim), jnp.float32),
            pltpu.VMEM((chunk,), jnp.int32),
            pltpu.VMEM((chunk, d_dim), jnp.float32),
            pltpu.SemaphoreType.DMA,
            pltpu.SemaphoreType.DMA,
        ],
    )
    def gather_kernel(emb_hbm, idx_hbm, out_hbm,
                      idx_a, rows_a, idx_b, rows_b, sem_a, sem_b):
        wid = lax.axis_index("s") * nc + lax.axis_index("c")
        base = wid * b_per_w
        pltpu.sync_copy(idx_hbm.at[pl.ds(base, chunk)], idx_a)
        pltpu.sync_copy(idx_hbm.at[pl.ds(base + chunk, chunk)], idx_b)
        cp_a = pltpu.async_copy(emb_hbm.at[idx_a], rows_a, sem_a)
        cp_b = pltpu.async_copy(emb_hbm.at[idx_b], rows_b, sem_b)
        cp_a.wait()
        pltpu.sync_copy(rows_a, out_hbm.at[pl.ds(base, chunk)])
        cp_b.wait()
        pltpu.sync_copy(rows_b, out_hbm.at[pl.ds(base + chunk, chunk)])

    return gather_kernel(embedding, ids_flat)


def kernel(z, embedding):
    b, d_dim, t = z.shape
    z_flat = jnp.transpose(z, (0, 2, 1)).reshape(b * t, d_dim)
    ids_3d, loss_arr = _tc_argmin(z_flat, embedding)
    ids_flat = ids_3d.reshape(b * t)
    zq_flat = _sc_gather(embedding, ids_flat)
    z_q = jnp.transpose(zq_flat.reshape(b, t, d_dim), (0, 2, 1))
    return (z_q, ids_flat.reshape(b, t), loss_arr[0, 0])


# speedup vs baseline: 1.4558x; 1.1051x over previous
import functools

import jax
import jax.numpy as jnp
from jax import lax
from jax.experimental import pallas as pl
from jax.experimental.pallas import tpu as pltpu
from jax.experimental.pallas import tpu_sc as plsc

CODEBOOK_SIZE = 8192
LATENT_DIM = 256
COMMITMENT_COST = 0.25

_KB = 2048
_NL = 128


def _argmin_body(z_ref, e_ref, ids_ref, loss_ref, en_ref):
    t = pl.program_id(0)
    nt = pl.num_programs(0)
    tb = z_ref.shape[2]
    z = jnp.transpose(z_ref[0], (1, 0))
    zn = jnp.sum(z * z, axis=1, keepdims=True)

    @pl.when(t == 0)
    def _en():
        for kb in range(CODEBOOK_SIZE // _KB):
            e = e_ref[pl.ds(kb * _KB, _KB), :]
            en_ref[0, pl.ds(kb * _KB, _KB)] = jnp.sum(e * e, axis=1)

    run_val = jnp.full((tb, _NL), jnp.inf, dtype=jnp.float32)
    run_vid = jnp.zeros((tb, _NL), dtype=jnp.int32)
    for kb in range(CODEBOOK_SIZE // _KB):
        e = e_ref[pl.ds(kb * _KB, _KB), :]
        en = en_ref[0, pl.ds(kb * _KB, _KB)]
        zw = lax.dot_general(z, e, (((1,), (1,)), ((), ())),
                             preferred_element_type=jnp.float32)
        d = (zn + en[None, :]) - 2.0 * zw
        for v in range(_KB // _NL):
            vg = kb * (_KB // _NL) + v
            sl = d[:, v * _NL:(v + 1) * _NL]
            lt = sl < run_val
            run_val = jnp.where(lt, sl, run_val)
            run_vid = jnp.where(lt, vg, run_vid)

    m = jnp.min(run_val, axis=1, keepdims=True)
    kidx = run_vid * _NL + lax.broadcasted_iota(jnp.int32, (tb, _NL), 1)
    pick = jnp.where(run_val == m, kidx, CODEBOOK_SIZE)
    idx = jnp.min(pick, axis=1)
    ids_ref[...] = idx.reshape(1, 1, tb)

    part = jnp.sum(m).reshape(1, 1)

    @pl.when(t == 0)
    def _init():
        loss_ref[...] = jnp.zeros((1, 1), jnp.float32)

    loss_ref[...] += part

    @pl.when(t == nt - 1)
    def _fin():
        n_elems = jnp.float32(nt * tb * LATENT_DIM)
        loss_ref[...] = loss_ref[...] * ((1.0 + COMMITMENT_COST) / n_elems)


def _tc_argmin(z, embedding):
    b, d_dim, t = z.shape
    return pl.pallas_call(
        _argmin_body,
        grid=(b,),
        in_specs=[
            pl.BlockSpec((1, d_dim, t), lambda i: (i, 0, 0)),
            pl.BlockSpec((CODEBOOK_SIZE, d_dim), lambda i: (0, 0)),
        ],
        out_specs=[
            pl.BlockSpec((1, 1, t), lambda i: (i, 0, 0)),
            pl.BlockSpec((1, 1), lambda i: (0, 0)),
        ],
        out_shape=[
            jax.ShapeDtypeStruct((b, 1, t), jnp.int32),
            jax.ShapeDtypeStruct((1, 1), jnp.float32),
        ],
        scratch_shapes=[pltpu.VMEM((1, CODEBOOK_SIZE), jnp.float32)],
    )(z, embedding)


def _sc_gather(embedding, ids_flat):
    n_tok = ids_flat.shape[0]
    d_dim = embedding.shape[1]
    info = plsc.get_sparse_core_info()
    nc, ns = info.num_cores, info.num_subcores
    nw = nc * ns
    b_per_w = n_tok // nw
    n_chunks = 2
    chunk = b_per_w // n_chunks
    mesh = plsc.VectorSubcoreMesh(core_axis_name="c", subcore_axis_name="s")

    @functools.partial(
        pl.kernel,
        mesh=mesh,
        out_type=jax.ShapeDtypeStruct((n_tok, d_dim), jnp.float32),
        scratch_types=[
            pltpu.VMEM((chunk,), jnp.int32),
            pltpu.VMEM((chunk, d_dim), jnp.float32),
            pltpu.VMEM((chunk,), jnp.int32),
            pltpu.VMEM((chunk, d_dim), jnp.float32),
            pltpu.SemaphoreType.DMA,
            pltpu.SemaphoreType.DMA,
        ],
    )
    def gather_kernel(emb_hbm, idx_hbm, out_hbm,
                      idx_a, rows_a, idx_b, rows_b, sem_a, sem_b):
        wid = lax.axis_index("s") * nc + lax.axis_index("c")
        base = wid * b_per_w
        pltpu.sync_copy(idx_hbm.at[pl.ds(base, chunk)], idx_a)
        pltpu.sync_copy(idx_hbm.at[pl.ds(base + chunk, chunk)], idx_b)
        cp_a = pltpu.async_copy(emb_hbm.at[idx_a], rows_a, sem_a)
        cp_b = pltpu.async_copy(emb_hbm.at[idx_b], rows_b, sem_b)
        cp_a.wait()
        pltpu.sync_copy(rows_a, out_hbm.at[pl.ds(base, chunk)])
        cp_b.wait()
        pltpu.sync_copy(rows_b, out_hbm.at[pl.ds(base + chunk, chunk)])

    return gather_kernel(embedding, ids_flat)


def kernel(z, embedding):
    b, d_dim, t = z.shape
    ids_3d, loss_arr = _tc_argmin(z, embedding)
    ids_flat = ids_3d.reshape(b * t)
    zq_flat = _sc_gather(embedding, ids_flat)
    z_q = jnp.transpose(zq_flat.reshape(b, t, d_dim), (0, 2, 1))
    return (z_q, ids_flat.reshape(b, t), loss_arr[0, 0])


# speedup vs baseline: 1.5809x; 1.0859x over previous
import functools

import jax
import jax.numpy as jnp
from jax import lax
from jax.experimental import pallas as pl
from jax.experimental.pallas import tpu as pltpu
from jax.experimental.pallas import tpu_sc as plsc

CODEBOOK_SIZE = 8192
LATENT_DIM = 256
COMMITMENT_COST = 0.25

_KB = 2048
_NL = 128


def _argmin_body(z_ref, e_ref, ids_ref, loss_ref, en_ref, e2_ref):
    t = pl.program_id(0)
    nt = pl.num_programs(0)
    tb = z_ref.shape[2]
    z = jnp.transpose(z_ref[0], (1, 0))
    zn = jnp.sum(z * z, axis=1, keepdims=True)

    @pl.when(t == 0)
    def _en():
        for kb in range(CODEBOOK_SIZE // _KB):
            e = e_ref[pl.ds(kb * _KB, _KB), :]
            en_ref[0, pl.ds(kb * _KB, _KB)] = jnp.sum(e * e, axis=1)
            e2_ref[pl.ds(kb * _KB, _KB), :] = e + e

    run_val = jnp.full((tb, _NL), jnp.inf, dtype=jnp.float32)
    run_vid = jnp.zeros((tb, _NL), dtype=jnp.int32)
    for kb in range(CODEBOOK_SIZE // _KB):
        e2 = e2_ref[pl.ds(kb * _KB, _KB), :]
        zw2 = lax.dot_general(z, e2, (((1,), (1,)), ((), ())),
                              preferred_element_type=jnp.float32)
        for v in range(_KB // _NL):
            vg = kb * (_KB // _NL) + v
            ken = kb * _KB + v * _NL
            en = en_ref[0, ken:ken + _NL]
            sl = (zn + en[None, :]) - zw2[:, v * _NL:(v + 1) * _NL]
            lt = sl < run_val
            run_val = jnp.where(lt, sl, run_val)
            run_vid = jnp.where(lt, vg, run_vid)

    m = jnp.min(run_val, axis=1, keepdims=True)
    kidx = run_vid * _NL + lax.broadcasted_iota(jnp.int32, (tb, _NL), 1)
    pick = jnp.where(run_val == m, kidx, CODEBOOK_SIZE)
    idx = jnp.min(pick, axis=1)
    ids_ref[...] = idx.reshape(1, 1, tb)

    part = jnp.sum(m).reshape(1, 1)

    @pl.when(t == 0)
    def _init():
        loss_ref[...] = jnp.zeros((1, 1), jnp.float32)

    loss_ref[...] += part

    @pl.when(t == nt - 1)
    def _fin():
        n_elems = jnp.float32(nt * tb * LATENT_DIM)
        loss_ref[...] = loss_ref[...] * ((1.0 + COMMITMENT_COST) / n_elems)


def _tc_argmin(z, embedding):
    b, d_dim, t = z.shape
    return pl.pallas_call(
        _argmin_body,
        grid=(b,),
        in_specs=[
            pl.BlockSpec((1, d_dim, t), lambda i: (i, 0, 0)),
            pl.BlockSpec((CODEBOOK_SIZE, d_dim), lambda i: (0, 0)),
        ],
        out_specs=[
            pl.BlockSpec((1, 1, t), lambda i: (i, 0, 0)),
            pl.BlockSpec((1, 1), lambda i: (0, 0)),
        ],
        out_shape=[
            jax.ShapeDtypeStruct((b, 1, t), jnp.int32),
            jax.ShapeDtypeStruct((1, 1), jnp.float32),
        ],
        scratch_shapes=[
            pltpu.VMEM((1, CODEBOOK_SIZE), jnp.float32),
            pltpu.VMEM((CODEBOOK_SIZE, d_dim), jnp.float32),
        ],
    )(z, embedding)


def _sc_gather(embedding, ids_flat):
    n_tok = ids_flat.shape[0]
    d_dim = embedding.shape[1]
    info = plsc.get_sparse_core_info()
    nc, ns = info.num_cores, info.num_subcores
    nw = nc * ns
    b_per_w = n_tok // nw
    n_chunks = 2
    chunk = b_per_w // n_chunks
    mesh = plsc.VectorSubcoreMesh(core_axis_name="c", subcore_axis_name="s")

    @functools.partial(
        pl.kernel,
        mesh=mesh,
        out_type=jax.ShapeDtypeStruct((n_tok, d_dim), jnp.float32),
        scratch_types=[
            pltpu.VMEM((chunk,), jnp.int32),
            pltpu.VMEM((chunk, d_dim), jnp.float32),
            pltpu.VMEM((chunk,), jnp.int32),
            pltpu.VMEM((chunk, d_dim), jnp.float32),
            pltpu.SemaphoreType.DMA,
            pltpu.SemaphoreType.DMA,
        ],
    )
    def gather_kernel(emb_hbm, idx_hbm, out_hbm,
                      idx_a, rows_a, idx_b, rows_b, sem_a, sem_b):
        wid = lax.axis_index("s") * nc + lax.axis_index("c")
        base = wid * b_per_w
        pltpu.sync_copy(idx_hbm.at[pl.ds(base, chunk)], idx_a)
        pltpu.sync_copy(idx_hbm.at[pl.ds(base + chunk, chunk)], idx_b)
        cp_a = pltpu.async_copy(emb_hbm.at[idx_a], rows_a, sem_a)
        cp_b = pltpu.async_copy(emb_hbm.at[idx_b], rows_b, sem_b)
        cp_a.wait()
        pltpu.sync_copy(rows_a, out_hbm.at[pl.ds(base, chunk)])
        cp_b.wait()
        pltpu.sync_copy(rows_b, out_hbm.at[pl.ds(base + chunk, chunk)])

    return gather_kernel(embedding, ids_flat)


def kernel(z, embedding):
    b, d_dim, t = z.shape
    ids_3d, loss_arr = _tc_argmin(z, embedding)
    ids_flat = ids_3d.reshape(b * t)
    zq_flat = _sc_gather(embedding, ids_flat)
    z_q = jnp.transpose(zq_flat.reshape(b, t, d_dim), (0, 2, 1))
    return (z_q, ids_flat.reshape(b, t), loss_arr[0, 0])


# speedup vs baseline: 1.5883x; 1.0046x over previous
import functools

import jax
import jax.numpy as jnp
from jax import lax
from jax.experimental import pallas as pl
from jax.experimental.pallas import tpu as pltpu
from jax.experimental.pallas import tpu_sc as plsc

CODEBOOK_SIZE = 8192
LATENT_DIM = 256
COMMITMENT_COST = 0.25

_KB = 2048
_NL = 128


def _argmin_body(z_ref, e_ref, ids_ref, loss_ref, en_ref, e2_ref):
    t = pl.program_id(0)
    nt = pl.num_programs(0)
    tb = z_ref.shape[2]
    z = jnp.transpose(z_ref[0], (1, 0))
    zn = jnp.sum(z * z, axis=1, keepdims=True)

    @pl.when(t == 0)
    def _en():
        for kb in range(CODEBOOK_SIZE // _KB):
            e = e_ref[pl.ds(kb * _KB, _KB), :]
            en_ref[0, pl.ds(kb * _KB, _KB)] = jnp.sum(e * e, axis=1)
            e2_ref[pl.ds(kb * _KB, _KB), :] = e + e

    def _dot(kb):
        e2 = e2_ref[pl.ds(kb * _KB, _KB), :]
        return lax.dot_general(z, e2, (((1,), (1,)), ((), ())),
                               preferred_element_type=jnp.float32)

    run_val = jnp.full((tb, _NL), jnp.inf, dtype=jnp.float32)
    run_vid = jnp.zeros((tb, _NL), dtype=jnp.int32)
    n_kb = CODEBOOK_SIZE // _KB
    zw2 = _dot(0)
    for kb in range(n_kb):
        zw2_next = _dot(kb + 1) if kb + 1 < n_kb else None
        for v in range(_KB // _NL):
            vg = kb * (_KB // _NL) + v
            ken = kb * _KB + v * _NL
            en = en_ref[0, ken:ken + _NL]
            sl = (zn + en[None, :]) - zw2[:, v * _NL:(v + 1) * _NL]
            lt = sl < run_val
            run_val = jnp.where(lt, sl, run_val)
            run_vid = jnp.where(lt, vg, run_vid)
        zw2 = zw2_next

    m = jnp.min(run_val, axis=1, keepdims=True)
    kidx = run_vid * _NL + lax.broadcasted_iota(jnp.int32, (tb, _NL), 1)
    pick = jnp.where(run_val == m, kidx, CODEBOOK_SIZE)
    idx = jnp.min(pick, axis=1)
    ids_ref[...] = idx.reshape(1, 1, tb)

    part = jnp.sum(m).reshape(1, 1)

    @pl.when(t == 0)
    def _init():
        loss_ref[...] = jnp.zeros((1, 1), jnp.float32)

    loss_ref[...] += part

    @pl.when(t == nt - 1)
    def _fin():
        n_elems = jnp.float32(nt * tb * LATENT_DIM)
        loss_ref[...] = loss_ref[...] * ((1.0 + COMMITMENT_COST) / n_elems)


def _tc_argmin(z, embedding):
    b, d_dim, t = z.shape
    return pl.pallas_call(
        _argmin_body,
        grid=(b,),
        in_specs=[
            pl.BlockSpec((1, d_dim, t), lambda i: (i, 0, 0)),
            pl.BlockSpec((CODEBOOK_SIZE, d_dim), lambda i: (0, 0)),
        ],
        out_specs=[
            pl.BlockSpec((1, 1, t), lambda i: (i, 0, 0)),
            pl.BlockSpec((1, 1), lambda i: (0, 0)),
        ],
        out_shape=[
            jax.ShapeDtypeStruct((b, 1, t), jnp.int32),
            jax.ShapeDtypeStruct((1, 1), jnp.float32),
        ],
        scratch_shapes=[
            pltpu.VMEM((1, CODEBOOK_SIZE), jnp.float32),
            pltpu.VMEM((CODEBOOK_SIZE, d_dim), jnp.float32),
        ],
    )(z, embedding)


def _sc_gather(embedding, ids_flat):
    n_tok = ids_flat.shape[0]
    d_dim = embedding.shape[1]
    info = plsc.get_sparse_core_info()
    nc, ns = info.num_cores, info.num_subcores
    nw = nc * ns
    b_per_w = n_tok // nw
    n_chunks = 2
    chunk = b_per_w // n_chunks
    mesh = plsc.VectorSubcoreMesh(core_axis_name="c", subcore_axis_name="s")

    @functools.partial(
        pl.kernel,
        mesh=mesh,
        out_type=jax.ShapeDtypeStruct((n_tok, d_dim), jnp.float32),
        scratch_types=[
            pltpu.VMEM((chunk,), jnp.int32),
            pltpu.VMEM((chunk, d_dim), jnp.float32),
            pltpu.VMEM((chunk,), jnp.int32),
            pltpu.VMEM((chunk, d_dim), jnp.float32),
            pltpu.SemaphoreType.DMA,
            pltpu.SemaphoreType.DMA,
        ],
    )
    def gather_kernel(emb_hbm, idx_hbm, out_hbm,
                      idx_a, rows_a, idx_b, rows_b, sem_a, sem_b):
        wid = lax.axis_index("s") * nc + lax.axis_index("c")
        base = wid * b_per_w
        pltpu.sync_copy(idx_hbm.at[pl.ds(base, chunk)], idx_a)
        pltpu.sync_copy(idx_hbm.at[pl.ds(base + chunk, chunk)], idx_b)
        cp_a = pltpu.async_copy(emb_hbm.at[idx_a], rows_a, sem_a)
        cp_b = pltpu.async_copy(emb_hbm.at[idx_b], rows_b, sem_b)
        cp_a.wait()
        pltpu.sync_copy(rows_a, out_hbm.at[pl.ds(base, chunk)])
        cp_b.wait()
        pltpu.sync_copy(rows_b, out_hbm.at[pl.ds(base + chunk, chunk)])

    return gather_kernel(embedding, ids_flat)


def kernel(z, embedding):
    b, d_dim, t = z.shape
    ids_3d, loss_arr = _tc_argmin(z, embedding)
    ids_flat = ids_3d.reshape(b * t)
    zq_flat = _sc_gather(embedding, ids_flat)
    z_q = jnp.transpose(zq_flat.reshape(b, t, d_dim), (0, 2, 1))
    return (z_q, ids_flat.reshape(b, t), loss_arr[0, 0])


# speedup vs baseline: 1.7457x; 1.0991x over previous
import functools

import jax
import jax.numpy as jnp
from jax import lax
from jax.experimental import pallas as pl
from jax.experimental.pallas import tpu as pltpu
from jax.experimental.pallas import tpu_sc as plsc

CODEBOOK_SIZE = 8192
LATENT_DIM = 256
COMMITMENT_COST = 0.25

_KB = 2048
_NL = 128


def _argmin_body(z_ref, e_ref, ids_ref, loss_ref, en_ref, e2_ref):
    t = pl.program_id(0)
    nt = pl.num_programs(0)
    tb = z_ref.shape[0]
    z = z_ref[...]
    zn = jnp.sum(z * z, axis=1, keepdims=True)

    @pl.when(t == 0)
    def _en():
        for kb in range(CODEBOOK_SIZE // _KB):
            e = e_ref[pl.ds(kb * _KB, _KB), :]
            en_ref[0, pl.ds(kb * _KB, _KB)] = jnp.sum(e * e, axis=1)
            e2_ref[pl.ds(kb * _KB, _KB), :] = e + e

    def _dot(kb):
        e2 = e2_ref[pl.ds(kb * _KB, _KB), :]
        return lax.dot_general(z, e2, (((1,), (1,)), ((), ())),
                               preferred_element_type=jnp.float32)

    run_val = jnp.full((tb, _NL), jnp.inf, dtype=jnp.float32)
    run_vid = jnp.zeros((tb, _NL), dtype=jnp.int32)
    n_kb = CODEBOOK_SIZE // _KB
    zw2 = _dot(0)
    for kb in range(n_kb):
        zw2_next = _dot(kb + 1) if kb + 1 < n_kb else None
        for v in range(_KB // _NL):
            vg = kb * (_KB // _NL) + v
            ken = kb * _KB + v * _NL
            en = en_ref[0, ken:ken + _NL]
            sl = (zn + en[None, :]) - zw2[:, v * _NL:(v + 1) * _NL]
            lt = sl < run_val
            run_val = jnp.where(lt, sl, run_val)
            run_vid = jnp.where(lt, vg, run_vid)
        zw2 = zw2_next

    m = jnp.min(run_val, axis=1, keepdims=True)
    kidx = run_vid * _NL + lax.broadcasted_iota(jnp.int32, (tb, _NL), 1)
    pick = jnp.where(run_val == m, kidx, CODEBOOK_SIZE)
    idx = jnp.min(pick, axis=1)
    ids_ref[...] = idx.reshape(1, 1, tb)

    part = jnp.sum(m).reshape(1, 1)

    @pl.when(t == 0)
    def _init():
        loss_ref[...] = jnp.zeros((1, 1), jnp.float32)

    loss_ref[...] += part

    @pl.when(t == nt - 1)
    def _fin():
        n_elems = jnp.float32(nt * tb * LATENT_DIM)
        loss_ref[...] = loss_ref[...] * ((1.0 + COMMITMENT_COST) / n_elems)


def _tc_argmin(z_flat, embedding, tb):
    n_tok, d_dim = z_flat.shape
    nt = n_tok // tb
    return pl.pallas_call(
        _argmin_body,
        grid=(nt,),
        in_specs=[
            pl.BlockSpec((tb, d_dim), lambda i: (i, 0)),
            pl.BlockSpec((CODEBOOK_SIZE, d_dim), lambda i: (0, 0)),
        ],
        out_specs=[
            pl.BlockSpec((1, 1, tb), lambda i: (i, 0, 0)),
            pl.BlockSpec((1, 1), lambda i: (0, 0)),
        ],
        out_shape=[
            jax.ShapeDtypeStruct((nt, 1, tb), jnp.int32),
            jax.ShapeDtypeStruct((1, 1), jnp.float32),
        ],
        scratch_shapes=[
            pltpu.VMEM((1, CODEBOOK_SIZE), jnp.float32),
            pltpu.VMEM((CODEBOOK_SIZE, d_dim), jnp.float32),
        ],
    )(z_flat, embedding)


def _sc_gather(embedding, ids_flat):
    n_tok = ids_flat.shape[0]
    d_dim = embedding.shape[1]
    info = plsc.get_sparse_core_info()
    nc, ns = info.num_cores, info.num_subcores
    nw = nc * ns
    b_per_w = n_tok // nw
    n_chunks = 2
    chunk = b_per_w // n_chunks
    mesh = plsc.VectorSubcoreMesh(core_axis_name="c", subcore_axis_name="s")

    @functools.partial(
        pl.kernel,
        mesh=mesh,
        out_type=jax.ShapeDtypeStruct((n_tok, d_dim), jnp.float32),
        scratch_types=[
            pltpu.VMEM((chunk,), jnp.int32),
            pltpu.VMEM((chunk, d_dim), jnp.float32),
            pltpu.VMEM((chunk,), jnp.int32),
            pltpu.VMEM((chunk, d_dim), jnp.float32),
            pltpu.SemaphoreType.DMA,
            pltpu.SemaphoreType.DMA,
        ],
    )
    def gather_kernel(emb_hbm, idx_hbm, out_hbm,
                      idx_a, rows_a, idx_b, rows_b, sem_a, sem_b):
        wid = lax.axis_index("s") * nc + lax.axis_index("c")
        base = wid * b_per_w
        pltpu.sync_copy(idx_hbm.at[pl.ds(base, chunk)], idx_a)
        pltpu.sync_copy(idx_hbm.at[pl.ds(base + chunk, chunk)], idx_b)
        cp_a = pltpu.async_copy(emb_hbm.at[idx_a], rows_a, sem_a)
        cp_b = pltpu.async_copy(emb_hbm.at[idx_b], rows_b, sem_b)
        cp_a.wait()
        pltpu.sync_copy(rows_a, out_hbm.at[pl.ds(base, chunk)])
        cp_b.wait()
        pltpu.sync_copy(rows_b, out_hbm.at[pl.ds(base + chunk, chunk)])

    return gather_kernel(embedding, ids_flat)


def kernel(z, embedding):
    b, d_dim, t = z.shape
    z_flat = jnp.transpose(z, (0, 2, 1)).reshape(b * t, d_dim)
    ids_3d, loss_arr = _tc_argmin(z_flat, embedding, t)
    ids_flat = ids_3d.reshape(b * t)
    zq_flat = _sc_gather(embedding, ids_flat)
    z_q = jnp.transpose(zq_flat.reshape(b, t, d_dim), (0, 2, 1))
    return (z_q, ids_flat.reshape(b, t), loss_arr[0, 0])
